# Initial kernel scaffold; baseline (speedup 1.0000x reference)
import jax
import jax.numpy as jnp
from jax.experimental import pallas as pl


def kernel(idx, targets, table):
    raise NotImplementedError("write your pallas kernel here")


# speedup vs baseline: 1.4451x; 1.4451x over previous
import functools

import jax
import jax.numpy as jnp
from jax import lax
from jax.experimental import pallas as pl
from jax.experimental.pallas import tpu as pltpu
from jax.experimental.pallas import tpu_sc as plsc

V = 1000
N = 1024 * 50
LANES = 16


def _row_stats_body(table_ref, out_ref):
    x = table_ref[...]
    m = jnp.max(x, axis=1, keepdims=True)
    s = jnp.sum(jnp.exp(x - m), axis=1, keepdims=True)
    out_ref[...] = m + jnp.log(s)


def _row_logz(table):
    return pl.pallas_call(
        _row_stats_body,
        out_shape=jax.ShapeDtypeStruct((V, 1), jnp.float32),
    )(table)


def _make_sc_gather(nc, ns, chunk):
    nw = nc * ns
    per_w = N // nw
    mesh = plsc.VectorSubcoreMesh(core_axis_name="c", subcore_axis_name="s",
                                  num_cores=nc, num_subcores=ns)

    @functools.partial(
        pl.kernel,
        out_type=(
            jax.ShapeDtypeStruct((N, V), jnp.float32),
            jax.ShapeDtypeStruct((nw, LANES), jnp.float32),
        ),
        mesh=mesh,
        compiler_params=pltpu.CompilerParams(needs_layout_passes=False,
                                             use_tc_tiling_on_sc=False),
        scratch_types=[
            pltpu.VMEM((per_w,), jnp.int32),
            pltpu.VMEM((per_w,), jnp.int32),
            pltpu.VMEM((V,), jnp.float32),
            pltpu.VMEM((chunk, V), jnp.float32),
            pltpu.VMEM((LANES,), jnp.float32),
            pltpu.SemaphoreType.DMA,
        ],
    )
    def sc_gather(table_hbm, idx_hbm, tgt_hbm, logz_hbm,
                  out_hbm, part_hbm,
                  idx_v, tgt_v, logz_v, rows_v, acc_v, sem):
        wid = lax.axis_index("s") * nc + lax.axis_index("c")
        base = wid * per_w
        pltpu.sync_copy(idx_hbm.at[pl.ds(base, per_w)], idx_v)
        pltpu.sync_copy(tgt_hbm.at[pl.ds(base, per_w)], tgt_v)
        pltpu.sync_copy(logz_hbm, logz_v)

        def chunk_body(c, acc):
            off = c * chunk
            pltpu.async_copy(table_hbm.at[idx_v.at[pl.ds(off, chunk)]],
                             rows_v, sem).wait()
            pltpu.sync_copy(rows_v, out_hbm.at[pl.ds(base + off, chunk)])

            def jbody(j, a):
                jo = off + j * LANES
                iv = idx_v[pl.ds(jo, LANES)]
                tv = tgt_v[pl.ds(jo, LANES)]
                lz = plsc.load_gather(logz_v, [iv])
                rsel = lax.iota(jnp.int32, LANES) + j * LANES
                pk = plsc.load_gather(rows_v, [rsel, tv])
                return a + lz - pk

            return lax.fori_loop(0, chunk // LANES, jbody, acc)

        acc = lax.fori_loop(0, per_w // chunk, chunk_body,
                            jnp.zeros((LANES,), jnp.float32))
        acc_v[...] = acc
        pltpu.sync_copy(acc_v, part_hbm.at[wid])

    return sc_gather


def kernel(idx, targets, table):
    idx_f = idx.reshape(N)
    tgt_f = targets.reshape(N)
    row_logz = _row_logz(table).reshape(V)
    info = plsc.get_sparse_core_info()
    sc_gather = _make_sc_gather(info.num_cores, info.num_subcores, chunk=64)
    logits, parts = sc_gather(table, idx_f, tgt_f, row_logz)
    loss = jnp.sum(parts) / N
    return logits, loss


# speedup vs baseline: 1.4800x; 1.0241x over previous
import functools

import jax
import jax.numpy as jnp
from jax import lax
from jax.experimental import pallas as pl
from jax.experimental.pallas import tpu as pltpu
from jax.experimental.pallas import tpu_sc as plsc

V = 1000
N = 1024 * 50
LANES = 16


def _row_stats_body(table_ref, out_ref):
    x = table_ref[...]
    m = jnp.max(x, axis=1, keepdims=True)
    s = jnp.sum(jnp.exp(x - m), axis=1, keepdims=True)
    out_ref[...] = m + jnp.log(s)


def _row_logz(table):
    return pl.pallas_call(
        _row_stats_body,
        out_shape=jax.ShapeDtypeStruct((V, 1), jnp.float32),
    )(table)


def _make_sc_gather(nc, ns, chunk):
    nw = nc * ns
    per_w = N // nw
    nchunks = per_w // chunk
    assert nchunks % 2 == 0 and chunk % LANES == 0
    mesh = plsc.VectorSubcoreMesh(core_axis_name="c", subcore_axis_name="s",
                                  num_cores=nc, num_subcores=ns)

    @functools.partial(
        pl.kernel,
        out_type=(
            jax.ShapeDtypeStruct((N, V), jnp.float32),
            jax.ShapeDtypeStruct((nw, LANES), jnp.float32),
        ),
        mesh=mesh,
        compiler_params=pltpu.CompilerParams(needs_layout_passes=False,
                                             use_tc_tiling_on_sc=False),
        scratch_types=[
            pltpu.VMEM((per_w,), jnp.int32),
            pltpu.VMEM((per_w,), jnp.int32),
            pltpu.VMEM((V,), jnp.float32),
            pltpu.VMEM((chunk, V), jnp.float32),
            pltpu.VMEM((chunk, V), jnp.float32),
            pltpu.VMEM((LANES,), jnp.float32),
            pltpu.SemaphoreType.DMA,
            pltpu.SemaphoreType.DMA,
        ],
    )
    def sc_gather(table_hbm, idx_hbm, tgt_hbm, logz_hbm,
                  out_hbm, part_hbm,
                  idx_v, tgt_v, logz_v, rows0, rows1, acc_v, sem0, sem1):
        bufs = (rows0, rows1)
        sems = (sem0, sem1)
        wid = lax.axis_index("s") * nc + lax.axis_index("c")
        base = wid * per_w
        pltpu.sync_copy(idx_hbm.at[pl.ds(base, per_w)], idx_v)
        pltpu.sync_copy(tgt_hbm.at[pl.ds(base, per_w)], tgt_v)
        pltpu.sync_copy(logz_hbm, logz_v)

        def start_gather(c, b):
            pltpu.async_copy(table_hbm.at[idx_v.at[pl.ds(c * chunk, chunk)]],
                             bufs[b], sems[b])

        start_gather(0, 0)

        def sub_iter(cidx, b, acc):
            @pl.when(cidx + 1 < nchunks)
            def _():
                start_gather(cidx + 1, 1 - b)

            pltpu.make_async_copy(out_hbm.at[pl.ds(0, chunk)],
                                  bufs[b], sems[b]).wait()

            def jbody(j, a):
                jo = cidx * chunk + j * LANES
                iv = idx_v[pl.ds(jo, LANES)]
                tv = tgt_v[pl.ds(jo, LANES)]
                lz = plsc.load_gather(logz_v, [iv])
                rsel = lax.iota(jnp.int32, LANES) + j * LANES
                pk = plsc.load_gather(bufs[b], [rsel, tv])
                return a + lz - pk

            acc = lax.fori_loop(0, chunk // LANES, jbody, acc)
            pltpu.sync_copy(bufs[b], out_hbm.at[pl.ds(base + cidx * chunk,
                                                      chunk)])
            return acc

        def outer(o, acc):
            acc = sub_iter(o * 2, 0, acc)
            acc = sub_iter(o * 2 + 1, 1, acc)
            return acc

        acc = lax.fori_loop(0, nchunks // 2, outer,
                            jnp.zeros((LANES,), jnp.float32))
        acc_v[...] = acc
        pltpu.sync_copy(acc_v, part_hbm.at[wid])

    return sc_gather


def kernel(idx, targets, table):
    idx_f = idx.reshape(N)
    tgt_f = targets.reshape(N)
    row_logz = _row_logz(table).reshape(V)
    info = plsc.get_sparse_core_info()
    sc_gather = _make_sc_gather(info.num_cores, info.num_subcores, chunk=32)
    logits, parts = sc_gather(table, idx_f, tgt_f, row_logz)
    loss = jnp.sum(parts) / N
    return logits, loss


# speedup vs baseline: 2.4424x; 1.6503x over previous
import functools

import jax
import jax.numpy as jnp
from jax import lax
from jax.experimental import pallas as pl
from jax.experimental.pallas import tpu as pltpu
from jax.experimental.pallas import tpu_sc as plsc

V = 1000
N = 1024 * 50
LANES = 16


def _row_stats_body(table_ref, out_ref):
    x = table_ref[...]
    m = jnp.max(x, axis=1, keepdims=True)
    s = jnp.sum(jnp.exp(x - m), axis=1, keepdims=True)
    out_ref[...] = m + jnp.log(s)


def _row_logz(table):
    return pl.pallas_call(
        _row_stats_body,
        out_shape=jax.ShapeDtypeStruct((V, 1), jnp.float32),
    )(table)


VP = 1024


def _make_sc_gather(nc, ns, chunk):
    nw = nc * ns
    per_w = N // nw
    nchunks = per_w // chunk
    assert nchunks % 2 == 0 and chunk % LANES == 0
    mesh = plsc.VectorSubcoreMesh(core_axis_name="c", subcore_axis_name="s",
                                  num_cores=nc, num_subcores=ns)

    @functools.partial(
        pl.kernel,
        out_type=(
            jax.ShapeDtypeStruct((N, VP), jnp.float32),
            jax.ShapeDtypeStruct((nw * LANES,), jnp.float32),
        ),
        mesh=mesh,
        compiler_params=pltpu.CompilerParams(needs_layout_passes=False,
                                             use_tc_tiling_on_sc=True),
        scratch_types=[
            pltpu.VMEM((per_w,), jnp.int32),
            pltpu.VMEM((per_w,), jnp.int32),
            pltpu.VMEM((V,), jnp.float32),
            pltpu.VMEM((chunk, VP), jnp.float32),
            pltpu.VMEM((chunk, VP), jnp.float32),
            pltpu.VMEM((LANES,), jnp.float32),
            pltpu.SemaphoreType.DMA,
            pltpu.SemaphoreType.DMA,
        ],
    )
    def sc_gather(table_hbm, idx_hbm, tgt_hbm, logz_hbm,
                  out_hbm, part_hbm,
                  idx_v, tgt_v, logz_v, rows0, rows1, acc_v, sem0, sem1):
        bufs = (rows0, rows1)
        sems = (sem0, sem1)
        wid = lax.axis_index("s") * nc + lax.axis_index("c")
        base = wid * per_w
        pltpu.sync_copy(idx_hbm.at[pl.ds(base, per_w)], idx_v)
        pltpu.sync_copy(tgt_hbm.at[pl.ds(base, per_w)], tgt_v)
        pltpu.sync_copy(logz_hbm, logz_v)

        def start_gather(c, b):
            pltpu.async_copy(table_hbm.at[idx_v.at[pl.ds(c * chunk, chunk)]],
                             bufs[b], sems[b])

        start_gather(0, 0)

        def sub_iter(cidx, b, acc):
            @pl.when(cidx + 1 < nchunks)
            def _():
                start_gather(cidx + 1, 1 - b)

            pltpu.make_async_copy(table_hbm.at[pl.ds(0, chunk)],
                                  bufs[b], sems[b]).wait()

            def jbody(j, a):
                jo = cidx * chunk + j * LANES
                iv = idx_v[pl.ds(jo, LANES)]
                tv = tgt_v[pl.ds(jo, LANES)]
                lz = plsc.load_gather(logz_v, [iv])
                rsel = lax.iota(jnp.int32, LANES) + j * LANES
                pk = plsc.load_gather(bufs[b], [rsel, tv])
                return a + lz - pk

            acc = lax.fori_loop(0, chunk // LANES, jbody, acc)
            pltpu.sync_copy(bufs[b],
                            out_hbm.at[pl.ds(base + cidx * chunk, chunk)])
            return acc

        def outer(o, acc):
            acc = sub_iter(o * 2, 0, acc)
            acc = sub_iter(o * 2 + 1, 1, acc)
            return acc

        acc = lax.fori_loop(0, nchunks // 2, outer,
                            jnp.zeros((LANES,), jnp.float32))
        acc_v[...] = acc
        pltpu.sync_copy(acc_v, part_hbm.at[pl.ds(wid * LANES, LANES)])

    return sc_gather


def kernel(idx, targets, table):
    idx_f = idx.reshape(N)
    tgt_f = targets.reshape(N)
    row_logz = _row_logz(table).reshape(V)
    table_p = jnp.pad(table, ((0, 0), (0, VP - V)))
    info = plsc.get_sparse_core_info()
    sc_gather = _make_sc_gather(info.num_cores, info.num_subcores, chunk=32)
    logits_p, parts = sc_gather(table_p, idx_f, tgt_f, row_logz)
    logits = logits_p[:, :V]
    loss = jnp.sum(parts) / N
    return logits, loss
